# Initial kernel scaffold; baseline (speedup 1.0000x reference)
import jax
import jax.numpy as jnp
from jax.experimental import pallas as pl


def kernel(x, edge_index, W1, b1, beta1, beta2, W2, b2):
    raise NotImplementedError("write your pallas kernel here")


# speedup vs baseline: 19.0632x; 19.0632x over previous
import dataclasses
import functools

import jax
import jax.numpy as jnp
from jax import lax
from jax.experimental import pallas as pl
from jax.experimental.pallas import tpu as pltpu
from jax.experimental.pallas import tpu_sc as plsc

N_PAD = 10240
F = 16
TW = 32
B = 128
NC, NS = 2, 16
NW = NC * NS


def _num_blocks(e_total):
    per_tile = -(-e_total // NW)
    return -(-per_tile // B)


def _tc_pre_body(x_ref, w1_ref, b1_ref, beta_ref, tab_ref, hnd_ref, n_real):
    x = x_ref[...]
    h = jnp.maximum(jnp.dot(x, w1_ref[...],
                            preferred_element_type=jnp.float32,
                            precision=lax.Precision.HIGHEST)
                    + b1_ref[...][None, :], 0.0)
    row = lax.broadcasted_iota(jnp.int32, (x.shape[0], 1), 0)
    h = jnp.where(row < n_real, h, 0.0)
    norm = jnp.sqrt(jnp.sum(h * h, axis=-1, keepdims=True))
    hn = h / jnp.maximum(norm, 1e-12)
    tab_ref[:, 0:F] = h
    tab_ref[:, F:TW] = hn * beta_ref[0]
    hnd_ref[...] = hn


def _tc_mid_body(part_ref, beta_ref, tab_ref, hnd_ref):
    feat = part_ref[0, :, 0:F] + part_ref[1, :, 0:F]
    den = part_ref[0, :, F] + part_ref[1, :, F]
    h = feat / (den + 1e-16)[:, None]
    norm = jnp.sqrt(jnp.sum(h * h, axis=-1, keepdims=True))
    hn = h / jnp.maximum(norm, 1e-12)
    tab_ref[:, 0:F] = h
    tab_ref[:, F:TW] = hn * beta_ref[0]
    hnd_ref[...] = hn


def _tc_post_body(part_ref, w2_ref, b2_ref, out_ref, n_real):
    feat = part_ref[0, 0:n_real, 0:F] + part_ref[1, 0:n_real, 0:F]
    den = part_ref[0, 0:n_real, F] + part_ref[1, 0:n_real, F]
    h = feat / (den + 1e-16)[:, None]
    logits = jnp.dot(h, w2_ref[...],
                     preferred_element_type=jnp.float32,
                     precision=lax.Precision.HIGHEST) + b2_ref[...][None, :]
    m = jnp.max(logits, axis=-1, keepdims=True)
    z = logits - m
    lse = jnp.log(jnp.sum(jnp.exp(z), axis=-1, keepdims=True))
    out_ref[...] = z - lse


def _tc_pre(x_pad, w1, b1, beta):
    n_real = 10000
    return pl.pallas_call(
        functools.partial(_tc_pre_body, n_real=n_real),
        out_shape=(jax.ShapeDtypeStruct((N_PAD, TW), jnp.float32),
                   jax.ShapeDtypeStruct((N_PAD, F), jnp.float32)),
    )(x_pad, w1, b1, beta)


def _tc_mid(part, beta):
    return pl.pallas_call(
        _tc_mid_body,
        out_shape=(jax.ShapeDtypeStruct((N_PAD, TW), jnp.float32),
                   jax.ShapeDtypeStruct((N_PAD, F), jnp.float32)),
    )(part, beta)


def _tc_post(part, w2, b2, n_real):
    return pl.pallas_call(
        functools.partial(_tc_post_body, n_real=n_real),
        out_shape=jax.ShapeDtypeStruct((n_real, w2.shape[1]), jnp.float32),
    )(part, w2, b2)


def _sc_prop(tab, hnd, srcp, dstp, n_blocks):
    mesh = plsc.VectorSubcoreMesh(core_axis_name="c", subcore_axis_name="s")
    rows_per_tile = N_PAD // NS

    cp = pltpu.CompilerParams()
    if "needs_layout_passes" in pltpu.CompilerParams.__dataclass_fields__:
        cp = dataclasses.replace(cp, needs_layout_passes=False)
    if "use_tc_tiling_on_sc" in pltpu.CompilerParams.__dataclass_fields__:
        cp = dataclasses.replace(cp, use_tc_tiling_on_sc=False)

    @functools.partial(
        pl.kernel,
        out_type=jax.ShapeDtypeStruct((NC, N_PAD, TW), jnp.float32),
        mesh=mesh,
        compiler_params=cp,
        scratch_types=[
            pltpu.VMEM((B,), jnp.int32),
            pltpu.VMEM((B,), jnp.int32),
            pltpu.VMEM((B, TW), jnp.float32),
            pltpu.VMEM((B, F), jnp.float32),
            pltpu.VMEM((B, TW), jnp.float32),
            pltpu.VMEM((B,), jnp.float32),
            pltpu.VMEM((16, TW), jnp.float32),
            pltpu.VMEM_SHARED((N_PAD, TW), jnp.float32),
            pltpu.SemaphoreType.DMA,
            pltpu.SemaphoreType.DMA,
        ],
    )
    def k(tab_hbm, hnd_hbm, src_hbm, dst_hbm, out_hbm,
          src_v, dst_v, rows_v, hnd_v, scaled_v, w_v, z_v, acc_sp,
          sem1, sem2):
        cid = lax.axis_index("c")
        tid = lax.axis_index("s")
        wid = tid * NC + cid

        zero16 = jnp.zeros((16,), jnp.float32)

        @pl.loop(0, 16)
        def _(i):
            z_v[i, pl.ds(0, 16)] = zero16
            z_v[i, pl.ds(16, 16)] = zero16

        @pl.loop(0, rows_per_tile // 16)
        def _(r):
            pltpu.sync_copy(z_v, acc_sp.at[pl.ds(tid * rows_per_tile + r * 16, 16)])

        plsc.subcore_barrier()

        e0mask = jnp.where(lax.iota(jnp.int32, 16) == 0, 1.0, 0.0)

        @pl.loop(0, n_blocks)
        def _(t):
            base = (wid * n_blocks + t) * B
            pltpu.sync_copy(src_hbm.at[pl.ds(base, B)], src_v)
            pltpu.sync_copy(dst_hbm.at[pl.ds(base, B)], dst_v)
            cp1 = pltpu.async_copy(tab_hbm.at[src_v], rows_v, sem1)
            cp2 = pltpu.async_copy(hnd_hbm.at[dst_v], hnd_v, sem2)
            cp1.wait()
            cp2.wait()

            for g in range(B // 16):
                r16 = lax.iota(jnp.int32, 16) + g * 16
                acc = jnp.zeros((16,), jnp.float32)
                for c in range(F):
                    s = plsc.load_gather(
                        rows_v, [r16, jnp.full((16,), F + c, jnp.int32)])
                    d = plsc.load_gather(
                        hnd_v, [r16, jnp.full((16,), c, jnp.int32)])
                    acc = acc + s * d
                w_v[pl.ds(g * 16, 16)] = jnp.exp(acc)

            @pl.loop(0, B)
            def _(j):
                wsp = plsc.load_gather(w_v, [jnp.full((16,), j, jnp.int32)])
                scaled_v[j, pl.ds(0, 16)] = rows_v[j, pl.ds(0, 16)] * wsp
                scaled_v[j, pl.ds(16, 16)] = wsp * e0mask

            pltpu.sync_copy(scaled_v, acc_sp.at[dst_v], add=True)

        plsc.subcore_barrier()

        pltpu.sync_copy(
            acc_sp.at[pl.ds(tid * rows_per_tile, rows_per_tile)],
            out_hbm.at[cid, pl.ds(tid * rows_per_tile, rows_per_tile)])

    return k(tab, hnd, srcp, dstp)


def kernel(x, edge_index, W1, b1, beta1, beta2, W2, b2):
    n = x.shape[0]
    e_total = edge_index.shape[1] + n
    n_blocks = _num_blocks(e_total)
    e_pad = n_blocks * B * NW

    loop = jnp.arange(n, dtype=edge_index.dtype)
    pad_e = e_pad - e_total
    src = jnp.concatenate([edge_index[0], loop,
                           jnp.zeros((pad_e,), edge_index.dtype)])
    dst = jnp.concatenate([edge_index[1], loop,
                           jnp.full((pad_e,), n, edge_index.dtype)])

    x_pad = jnp.pad(x, ((0, N_PAD - n), (0, 0)))
    beta1v = jnp.reshape(beta1.astype(jnp.float32), (1,))
    beta2v = jnp.reshape(beta2.astype(jnp.float32), (1,))

    tab1, hnd1 = _tc_pre(x_pad, W1, b1, beta1v)
    part1 = _sc_prop(tab1, hnd1, src, dst, n_blocks)
    tab2, hnd2 = _tc_mid(part1, beta2v)
    part2 = _sc_prop(tab2, hnd2, src, dst, n_blocks)
    return _tc_post(part2, W2, b2, n)


# speedup vs baseline: 19.1481x; 1.0045x over previous
import dataclasses
import functools

import jax
import jax.numpy as jnp
from jax import lax
from jax.experimental import pallas as pl
from jax.experimental.pallas import tpu as pltpu
from jax.experimental.pallas import tpu_sc as plsc

N_PAD = 10240
F = 16
TW = 32
B = 128
NC, NS = 2, 16
NW = NC * NS


def _num_blocks(e_total):
    per_tile = -(-e_total // NW)
    nb = -(-per_tile // B)
    return nb + (nb & 1)


def _tc_pre_body(x_ref, w1_ref, b1_ref, beta_ref, tab_ref, hnd_ref, n_real):
    x = x_ref[...]
    h = jnp.maximum(jnp.dot(x, w1_ref[...],
                            preferred_element_type=jnp.float32,
                            precision=lax.Precision.HIGHEST)
                    + b1_ref[...][None, :], 0.0)
    row = lax.broadcasted_iota(jnp.int32, (x.shape[0], 1), 0)
    h = jnp.where(row < n_real, h, 0.0)
    norm = jnp.sqrt(jnp.sum(h * h, axis=-1, keepdims=True))
    hn = h / jnp.maximum(norm, 1e-12)
    tab_ref[:, 0:F] = h
    tab_ref[:, F:TW] = hn * beta_ref[0]
    hnd_ref[...] = hn


def _tc_mid_body(part_ref, beta_ref, tab_ref, hnd_ref):
    feat = part_ref[0, :, 0:F] + part_ref[1, :, 0:F]
    den = part_ref[0, :, F] + part_ref[1, :, F]
    h = feat / (den + 1e-16)[:, None]
    norm = jnp.sqrt(jnp.sum(h * h, axis=-1, keepdims=True))
    hn = h / jnp.maximum(norm, 1e-12)
    tab_ref[:, 0:F] = h
    tab_ref[:, F:TW] = hn * beta_ref[0]
    hnd_ref[...] = hn


def _tc_post_body(part_ref, w2_ref, b2_ref, out_ref, n_real):
    feat = part_ref[0, 0:n_real, 0:F] + part_ref[1, 0:n_real, 0:F]
    den = part_ref[0, 0:n_real, F] + part_ref[1, 0:n_real, F]
    h = feat / (den + 1e-16)[:, None]
    logits = jnp.dot(h, w2_ref[...],
                     preferred_element_type=jnp.float32,
                     precision=lax.Precision.HIGHEST) + b2_ref[...][None, :]
    m = jnp.max(logits, axis=-1, keepdims=True)
    z = logits - m
    lse = jnp.log(jnp.sum(jnp.exp(z), axis=-1, keepdims=True))
    out_ref[...] = z - lse


def _tc_pre(x_pad, w1, b1, beta):
    n_real = 10000
    return pl.pallas_call(
        functools.partial(_tc_pre_body, n_real=n_real),
        out_shape=(jax.ShapeDtypeStruct((N_PAD, TW), jnp.float32),
                   jax.ShapeDtypeStruct((N_PAD, F), jnp.float32)),
    )(x_pad, w1, b1, beta)


def _tc_mid(part, beta):
    return pl.pallas_call(
        _tc_mid_body,
        out_shape=(jax.ShapeDtypeStruct((N_PAD, TW), jnp.float32),
                   jax.ShapeDtypeStruct((N_PAD, F), jnp.float32)),
    )(part, beta)


def _tc_post(part, w2, b2, n_real):
    return pl.pallas_call(
        functools.partial(_tc_post_body, n_real=n_real),
        out_shape=jax.ShapeDtypeStruct((n_real, w2.shape[1]), jnp.float32),
    )(part, w2, b2)


def _sc_prop(tab, hnd, srcp, dstp, n_blocks):
    mesh = plsc.VectorSubcoreMesh(core_axis_name="c", subcore_axis_name="s")
    rows_per_tile = N_PAD // NS

    cp = pltpu.CompilerParams()
    if "needs_layout_passes" in pltpu.CompilerParams.__dataclass_fields__:
        cp = dataclasses.replace(cp, needs_layout_passes=False)
    if "use_tc_tiling_on_sc" in pltpu.CompilerParams.__dataclass_fields__:
        cp = dataclasses.replace(cp, use_tc_tiling_on_sc=False)

    @functools.partial(
        pl.kernel,
        out_type=jax.ShapeDtypeStruct((NC, N_PAD, TW), jnp.float32),
        mesh=mesh,
        compiler_params=cp,
        scratch_types=[
            pltpu.VMEM((n_blocks, B), jnp.int32),
            pltpu.VMEM((n_blocks, B), jnp.int32),
            pltpu.VMEM((2, B, TW), jnp.float32),
            pltpu.VMEM((2, B, F), jnp.float32),
            pltpu.VMEM((2, B, TW), jnp.float32),
            pltpu.VMEM((16, TW), jnp.float32),
            pltpu.VMEM_SHARED((N_PAD, TW), jnp.float32),
            pltpu.SemaphoreType.DMA,
            pltpu.SemaphoreType.DMA,
            pltpu.SemaphoreType.DMA,
            pltpu.SemaphoreType.DMA,
            pltpu.SemaphoreType.DMA,
            pltpu.SemaphoreType.DMA,
            pltpu.SemaphoreType.DMA,
        ],
    )
    def k(tab_hbm, hnd_hbm, src_hbm, dst_hbm, out_hbm,
          src_a, dst_a, rows_v, hnd_v, scaled_v, z_v, acc_sp,
          sem_i, sem_r0, sem_r1, sem_h0, sem_h1, sem_s0, sem_s1):
        cid = lax.axis_index("c")
        tid = lax.axis_index("s")
        wid = tid * NC + cid
        sem_r = (sem_r0, sem_r1)
        sem_h = (sem_h0, sem_h1)
        sem_s = (sem_s0, sem_s1)

        zero16 = jnp.zeros((16,), jnp.float32)

        cpi1 = pltpu.async_copy(src_hbm.at[wid], src_a, sem_i)
        cpi2 = pltpu.async_copy(dst_hbm.at[wid], dst_a, sem_i)

        @pl.loop(0, 16)
        def _(i):
            z_v[i, pl.ds(0, 16)] = zero16
            z_v[i, pl.ds(16, 16)] = zero16

        @pl.loop(0, rows_per_tile // 16)
        def _(r):
            pltpu.sync_copy(z_v, acc_sp.at[pl.ds(tid * rows_per_tile + r * 16, 16)])

        for b in range(2):
            @pl.loop(0, B)
            def _(r):
                scaled_v.at[b][r, pl.ds(F, F)] = zero16

        cpi1.wait()
        cpi2.wait()
        plsc.subcore_barrier()

        def issue(t, b):
            pltpu.async_copy(tab_hbm.at[src_a.at[t]], rows_v.at[b], sem_r[b])
            pltpu.async_copy(hnd_hbm.at[dst_a.at[t]], hnd_v.at[b], sem_h[b])

        def wait_gather(t, b):
            pltpu.make_async_copy(
                tab_hbm.at[src_a.at[t]], rows_v.at[b], sem_r[b]).wait()
            pltpu.make_async_copy(
                hnd_hbm.at[dst_a.at[t]], hnd_v.at[b], sem_h[b]).wait()

        def wait_scatter(t, b):
            pltpu.make_async_copy(
                scaled_v.at[b], acc_sp.at[dst_a.at[t]], sem_s[b]).wait()

        def compute(t, b):
            rows = rows_v.at[b]
            hnd = hnd_v.at[b]
            scaled = scaled_v.at[b]
            for g in range(B // 16):
                r16 = lax.iota(jnp.int32, 16) + g * 16
                acc = jnp.zeros((16,), jnp.float32)
                for c in range(F):
                    s = plsc.load_gather(
                        rows, [r16, jnp.full((16,), F + c, jnp.int32)])
                    d = plsc.load_gather(
                        hnd, [r16, jnp.full((16,), c, jnp.int32)])
                    acc = acc + s * d
                w = jnp.exp(acc)
                plsc.store_scatter(
                    scaled, [r16, jnp.full((16,), F, jnp.int32)], w)
                for c in range(F):
                    hcomp = plsc.load_gather(
                        rows, [r16, jnp.full((16,), c, jnp.int32)])
                    plsc.store_scatter(
                        scaled, [r16, jnp.full((16,), c, jnp.int32)],
                        hcomp * w)
            pltpu.async_copy(scaled, acc_sp.at[dst_a.at[t]], sem_s[b],
                             add=True)

        issue(0, 0)

        @pl.loop(0, n_blocks, step=2)
        def _(tt):
            issue(tt + 1, 1)
            wait_gather(tt, 0)

            @pl.when(tt >= 2)
            def _():
                wait_scatter(tt - 2, 0)
            compute(tt, 0)

            @pl.when(tt + 2 < n_blocks)
            def _():
                issue(tt + 2, 0)
            wait_gather(tt + 1, 1)

            @pl.when(tt >= 2)
            def _():
                wait_scatter(tt - 1, 1)
            compute(tt + 1, 1)

        wait_scatter(n_blocks - 2, 0)
        wait_scatter(n_blocks - 1, 1)
        plsc.subcore_barrier()

        pltpu.sync_copy(
            acc_sp.at[pl.ds(tid * rows_per_tile, rows_per_tile)],
            out_hbm.at[cid, pl.ds(tid * rows_per_tile, rows_per_tile)])

    return k(tab, hnd, srcp, dstp)


def kernel(x, edge_index, W1, b1, beta1, beta2, W2, b2):
    n = x.shape[0]
    e_total = edge_index.shape[1] + n
    n_blocks = _num_blocks(e_total)
    e_pad = n_blocks * B * NW

    loop = jnp.arange(n, dtype=edge_index.dtype)
    pad_e = e_pad - e_total
    src = jnp.concatenate([edge_index[0], loop,
                           jnp.zeros((pad_e,), edge_index.dtype)])
    dst = jnp.concatenate([edge_index[1], loop,
                           jnp.full((pad_e,), n, edge_index.dtype)])
    src = src.reshape(NW, n_blocks, B)
    dst = dst.reshape(NW, n_blocks, B)

    x_pad = jnp.pad(x, ((0, N_PAD - n), (0, 0)))
    beta1v = jnp.reshape(beta1.astype(jnp.float32), (1,))
    beta2v = jnp.reshape(beta2.astype(jnp.float32), (1,))

    tab1, hnd1 = _tc_pre(x_pad, W1, b1, beta1v)
    part1 = _sc_prop(tab1, hnd1, src, dst, n_blocks)
    tab2, hnd2 = _tc_mid(part1, beta2v)
    part2 = _sc_prop(tab2, hnd2, src, dst, n_blocks)
    return _tc_post(part2, W2, b2, n)


# speedup vs baseline: 26.9795x; 1.4090x over previous
import dataclasses
import functools

import jax
import jax.numpy as jnp
from jax import lax
from jax.experimental import pallas as pl
from jax.experimental.pallas import tpu as pltpu
from jax.experimental.pallas import tpu_sc as plsc

_DIAG = ""

N_PAD = 10240
F = 16
TW = 32
B = 128
NC, NS = 2, 16
NW = NC * NS


def _num_blocks(e_total):
    per_tile = -(-e_total // NW)
    nb = -(-per_tile // B)
    return nb + (nb & 1)


def _tc_pre_body(x_ref, w1_ref, b1_ref, beta_ref, tab_ref, hnd_ref, n_real):
    x = x_ref[...]
    h = jnp.maximum(jnp.dot(x, w1_ref[...],
                            preferred_element_type=jnp.float32,
                            precision=lax.Precision.HIGHEST)
                    + b1_ref[...][None, :], 0.0)
    row = lax.broadcasted_iota(jnp.int32, (x.shape[0], 1), 0)
    h = jnp.where(row < n_real, h, 0.0)
    norm = jnp.sqrt(jnp.sum(h * h, axis=-1, keepdims=True))
    hn = h / jnp.maximum(norm, 1e-12)
    tab_ref[:, 0:F] = h
    tab_ref[:, F:TW] = hn * beta_ref[0]
    hnd_ref[...] = hn


def _tc_mid_body(part_ref, beta_ref, tab_ref, hnd_ref):
    feat = part_ref[0, :, 0:F] + part_ref[1, :, 0:F]
    den = part_ref[0, :, F] + part_ref[1, :, F]
    h = feat / (den + 1e-16)[:, None]
    norm = jnp.sqrt(jnp.sum(h * h, axis=-1, keepdims=True))
    hn = h / jnp.maximum(norm, 1e-12)
    tab_ref[:, 0:F] = h
    tab_ref[:, F:TW] = hn * beta_ref[0]
    hnd_ref[...] = hn


def _tc_post_body(part_ref, w2_ref, b2_ref, out_ref, n_real):
    feat = part_ref[0, 0:n_real, 0:F] + part_ref[1, 0:n_real, 0:F]
    den = part_ref[0, 0:n_real, F] + part_ref[1, 0:n_real, F]
    h = feat / (den + 1e-16)[:, None]
    logits = jnp.dot(h, w2_ref[...],
                     preferred_element_type=jnp.float32,
                     precision=lax.Precision.HIGHEST) + b2_ref[...][None, :]
    m = jnp.max(logits, axis=-1, keepdims=True)
    z = logits - m
    lse = jnp.log(jnp.sum(jnp.exp(z), axis=-1, keepdims=True))
    out_ref[...] = z - lse


def _tc_pre(x_pad, w1, b1, beta):
    n_real = 10000
    return pl.pallas_call(
        functools.partial(_tc_pre_body, n_real=n_real),
        out_shape=(jax.ShapeDtypeStruct((N_PAD, TW), jnp.float32),
                   jax.ShapeDtypeStruct((N_PAD, F), jnp.float32)),
    )(x_pad, w1, b1, beta)


def _tc_mid(part, beta):
    return pl.pallas_call(
        _tc_mid_body,
        out_shape=(jax.ShapeDtypeStruct((N_PAD, TW), jnp.float32),
                   jax.ShapeDtypeStruct((N_PAD, F), jnp.float32)),
    )(part, beta)


def _tc_post(part, w2, b2, n_real):
    return pl.pallas_call(
        functools.partial(_tc_post_body, n_real=n_real),
        out_shape=jax.ShapeDtypeStruct((n_real, w2.shape[1]), jnp.float32),
    )(part, w2, b2)


def _sc_prop(tab, hnd, srcp, dstp, n_blocks):
    mesh = plsc.VectorSubcoreMesh(core_axis_name="c", subcore_axis_name="s")
    rows_per_tile = N_PAD // NS

    cp = pltpu.CompilerParams()
    if "needs_layout_passes" in pltpu.CompilerParams.__dataclass_fields__:
        cp = dataclasses.replace(cp, needs_layout_passes=False)
    if "use_tc_tiling_on_sc" in pltpu.CompilerParams.__dataclass_fields__:
        cp = dataclasses.replace(cp, use_tc_tiling_on_sc=False)

    @functools.partial(
        pl.kernel,
        out_type=jax.ShapeDtypeStruct((NC, N_PAD, TW), jnp.float32),
        mesh=mesh,
        compiler_params=cp,
        scratch_types=[
            pltpu.VMEM((n_blocks, B), jnp.int32),
            pltpu.VMEM((n_blocks, B), jnp.int32),
            pltpu.VMEM((2, B, TW), jnp.float32),
            pltpu.VMEM((2, B, F), jnp.float32),
            pltpu.VMEM((2, B, TW), jnp.float32),
            pltpu.VMEM((16, TW), jnp.float32),
            pltpu.VMEM_SHARED((N_PAD, TW), jnp.float32),
            pltpu.SemaphoreType.DMA,
            pltpu.SemaphoreType.DMA,
            pltpu.SemaphoreType.DMA,
            pltpu.SemaphoreType.DMA,
            pltpu.SemaphoreType.DMA,
            pltpu.SemaphoreType.DMA,
            pltpu.SemaphoreType.DMA,
        ],
    )
    def k(tab_hbm, hnd_hbm, src_hbm, dst_hbm, out_hbm,
          src_a, dst_a, rows_v, hnd_v, scaled_v, z_v, acc_sp,
          sem_i, sem_r0, sem_r1, sem_h0, sem_h1, sem_s0, sem_s1):
        cid = lax.axis_index("c")
        tid = lax.axis_index("s")
        wid = tid * NC + cid
        sem_r = (sem_r0, sem_r1)
        sem_h = (sem_h0, sem_h1)
        sem_s = (sem_s0, sem_s1)

        zero16 = jnp.zeros((16,), jnp.float32)

        cpi1 = pltpu.async_copy(src_hbm.at[wid], src_a, sem_i)
        cpi2 = pltpu.async_copy(dst_hbm.at[wid], dst_a, sem_i)

        @pl.loop(0, 16)
        def _(i):
            z_v[i, pl.ds(0, 16)] = zero16
            z_v[i, pl.ds(16, 16)] = zero16

        @pl.loop(0, rows_per_tile // 16)
        def _(r):
            pltpu.sync_copy(z_v, acc_sp.at[pl.ds(tid * rows_per_tile + r * 16, 16)])

        for b in range(2):
            @pl.loop(0, B)
            def _(r):
                scaled_v.at[b][r, pl.ds(F, F)] = zero16

        cpi1.wait()
        cpi2.wait()
        plsc.subcore_barrier()

        def issue(t, b):
            if _DIAG == "nogather":
                return
            pltpu.async_copy(tab_hbm.at[src_a.at[t]], rows_v.at[b], sem_r[b])
            pltpu.async_copy(hnd_hbm.at[dst_a.at[t]], hnd_v.at[b], sem_h[b])

        def wait_gather(t, b):
            if _DIAG == "nogather":
                return
            pltpu.make_async_copy(
                tab_hbm.at[src_a.at[t]], rows_v.at[b], sem_r[b]).wait()
            pltpu.make_async_copy(
                hnd_hbm.at[dst_a.at[t]], hnd_v.at[b], sem_h[b]).wait()

        def wait_scatter(t, b):
            pltpu.make_async_copy(
                scaled_v.at[b], acc_sp.at[dst_a.at[t]], sem_s[b]).wait()

        def compute(t, b):
            rows = rows_v.at[b]
            hnd = hnd_v.at[b]
            scaled = scaled_v.at[b]

            @plsc.parallel_loop(0, B // 16, unroll=4)
            def _(g):
                r16 = lax.iota(jnp.int32, 16) + g * 16
                acc = jnp.zeros((16,), jnp.float32)
                for c in range(F):
                    s = plsc.load_gather(
                        rows, [r16, jnp.full((16,), F + c, jnp.int32)])
                    d = plsc.load_gather(
                        hnd, [r16, jnp.full((16,), c, jnp.int32)])
                    acc = acc + s * d
                w = jnp.exp(acc)
                plsc.store_scatter(
                    scaled, [r16, jnp.full((16,), F, jnp.int32)], w)
                for c in range(F):
                    hcomp = plsc.load_gather(
                        rows, [r16, jnp.full((16,), c, jnp.int32)])
                    plsc.store_scatter(
                        scaled, [r16, jnp.full((16,), c, jnp.int32)],
                        hcomp * w)

            if _DIAG != "noscatter":
                pltpu.async_copy(scaled, acc_sp.at[dst_a.at[t]], sem_s[b],
                                 add=True)

        issue(0, 0)

        @pl.loop(0, n_blocks, step=2)
        def _(tt):
            issue(tt + 1, 1)
            wait_gather(tt, 0)

            if _DIAG != "noscatter":
                @pl.when(tt >= 2)
                def _():
                    wait_scatter(tt - 2, 0)
            compute(tt, 0)

            @pl.when(tt + 2 < n_blocks)
            def _():
                issue(tt + 2, 0)
            wait_gather(tt + 1, 1)

            if _DIAG != "noscatter":
                @pl.when(tt >= 2)
                def _():
                    wait_scatter(tt - 1, 1)
            compute(tt + 1, 1)

        if _DIAG != "noscatter":
            wait_scatter(n_blocks - 2, 0)
            wait_scatter(n_blocks - 1, 1)
        plsc.subcore_barrier()

        pltpu.sync_copy(
            acc_sp.at[pl.ds(tid * rows_per_tile, rows_per_tile)],
            out_hbm.at[cid, pl.ds(tid * rows_per_tile, rows_per_tile)])

    return k(tab, hnd, srcp, dstp)


def kernel(x, edge_index, W1, b1, beta1, beta2, W2, b2):
    n = x.shape[0]
    e_total = edge_index.shape[1] + n
    n_blocks = _num_blocks(e_total)
    e_pad = n_blocks * B * NW

    loop = jnp.arange(n, dtype=edge_index.dtype)
    pad_e = e_pad - e_total
    src = jnp.concatenate([edge_index[0], loop,
                           jnp.zeros((pad_e,), edge_index.dtype)])
    dst = jnp.concatenate([edge_index[1], loop,
                           jnp.full((pad_e,), n, edge_index.dtype)])
    src = src.reshape(NW, n_blocks, B)
    dst = dst.reshape(NW, n_blocks, B)

    x_pad = jnp.pad(x, ((0, N_PAD - n), (0, 0)))
    beta1v = jnp.reshape(beta1.astype(jnp.float32), (1,))
    beta2v = jnp.reshape(beta2.astype(jnp.float32), (1,))

    tab1, hnd1 = _tc_pre(x_pad, W1, b1, beta1v)
    part1 = _sc_prop(tab1, hnd1, src, dst, n_blocks)
    tab2, hnd2 = _tc_mid(part1, beta2v)
    part2 = _sc_prop(tab2, hnd2, src, dst, n_blocks)
    return _tc_post(part2, W2, b2, n)


# speedup vs baseline: 34.1566x; 1.2660x over previous
import dataclasses
import functools

import jax
import jax.numpy as jnp
from jax import lax
from jax.experimental import pallas as pl
from jax.experimental.pallas import tpu as pltpu
from jax.experimental.pallas import tpu_sc as plsc

N_PAD = 10240
N_TAB = 10016
F = 16
TW = 32
B = 128
NC, NS = 2, 16
NW = NC * NS


def _num_blocks(e_total):
    per_tile = -(-e_total // NW)
    nb = -(-per_tile // B)
    return nb + (nb & 1)


def _tc_pre_body(x_ref, w1_ref, b1_ref, h_ref, hn_ref, n_real):
    x = x_ref[...]
    h = jnp.maximum(jnp.dot(x, w1_ref[...],
                            preferred_element_type=jnp.float32,
                            precision=lax.Precision.HIGHEST)
                    + b1_ref[...][None, :], 0.0)
    row = lax.broadcasted_iota(jnp.int32, (x.shape[0], 1), 0)
    h = jnp.where(row < n_real, h, 0.0)
    norm = jnp.sqrt(jnp.sum(h * h, axis=-1, keepdims=True))
    hn = h / jnp.maximum(norm, 1e-12)
    h_ref[...] = h
    hn_ref[...] = hn


def _tc_mid_body(part_ref, h_ref, hn_ref):
    feat = part_ref[0, :, 0:F] + part_ref[1, :, 0:F]
    den = part_ref[0, :, F] + part_ref[1, :, F]
    h = feat / (den + 1e-16)[:, None]
    norm = jnp.sqrt(jnp.sum(h * h, axis=-1, keepdims=True))
    hn = h / jnp.maximum(norm, 1e-12)
    h_ref[...] = h
    hn_ref[...] = hn


def _tc_post_body(part_ref, w2_ref, b2_ref, out_ref, n_real):
    feat = part_ref[0, 0:n_real, 0:F] + part_ref[1, 0:n_real, 0:F]
    den = part_ref[0, 0:n_real, F] + part_ref[1, 0:n_real, F]
    h = feat / (den + 1e-16)[:, None]
    logits = jnp.dot(h, w2_ref[...],
                     preferred_element_type=jnp.float32,
                     precision=lax.Precision.HIGHEST) + b2_ref[...][None, :]
    m = jnp.max(logits, axis=-1, keepdims=True)
    z = logits - m
    lse = jnp.log(jnp.sum(jnp.exp(z), axis=-1, keepdims=True))
    out_ref[...] = z - lse


def _tc_pre(x_pad, w1, b1):
    return pl.pallas_call(
        functools.partial(_tc_pre_body, n_real=10000),
        out_shape=(jax.ShapeDtypeStruct((N_PAD, F), jnp.float32),
                   jax.ShapeDtypeStruct((N_PAD, F), jnp.float32)),
    )(x_pad, w1, b1)


def _tc_mid(part):
    return pl.pallas_call(
        _tc_mid_body,
        out_shape=(jax.ShapeDtypeStruct((N_PAD, F), jnp.float32),
                   jax.ShapeDtypeStruct((N_PAD, F), jnp.float32)),
    )(part)


def _tc_post(part, w2, b2, n_real):
    return pl.pallas_call(
        functools.partial(_tc_post_body, n_real=n_real),
        out_shape=jax.ShapeDtypeStruct((n_real, w2.shape[1]), jnp.float32),
    )(part, w2, b2)


def _pack_pairs(hn):
    hn_bf = hn.astype(jnp.bfloat16)
    return jax.lax.bitcast_convert_type(
        hn_bf.reshape(N_PAD, F // 2, 2), jnp.float32)


def _sc_prop(htab, hnpk, bvec, srcp, dstp, n_blocks):
    mesh = plsc.VectorSubcoreMesh(core_axis_name="c", subcore_axis_name="s")
    rows_per_tile = N_PAD // NS

    cp = pltpu.CompilerParams()
    if "needs_layout_passes" in pltpu.CompilerParams.__dataclass_fields__:
        cp = dataclasses.replace(cp, needs_layout_passes=False)
    if "use_tc_tiling_on_sc" in pltpu.CompilerParams.__dataclass_fields__:
        cp = dataclasses.replace(cp, use_tc_tiling_on_sc=False)

    @functools.partial(
        pl.kernel,
        out_type=jax.ShapeDtypeStruct((NC, N_PAD, TW), jnp.float32),
        mesh=mesh,
        compiler_params=cp,
        scratch_types=[
            pltpu.VMEM((n_blocks, B), jnp.int32),
            pltpu.VMEM((n_blocks, B), jnp.int32),
            pltpu.VMEM((N_TAB, F // 2), jnp.float32),
            pltpu.VMEM((16,), jnp.float32),
            pltpu.VMEM((2, B, F), jnp.float32),
            pltpu.VMEM((B, TW), jnp.float32),
            pltpu.VMEM((16, TW), jnp.float32),
            pltpu.VMEM_SHARED((N_PAD, TW), jnp.float32),
            pltpu.SemaphoreType.DMA,
            pltpu.SemaphoreType.DMA,
            pltpu.SemaphoreType.DMA,
            pltpu.SemaphoreType.DMA,
        ],
    )
    def k(htab_hbm, hnpk_hbm, bvec_hbm, src_hbm, dst_hbm, out_hbm,
          src_a, dst_a, hnpk_v, beta_v, rows_v, scaled_v, z_v, acc_sp,
          sem_i, sem_r0, sem_r1, sem_s):
        cid = lax.axis_index("c")
        tid = lax.axis_index("s")
        wid = tid * NC + cid
        sem_r = (sem_r0, sem_r1)

        zero16 = jnp.zeros((16,), jnp.float32)

        cpi1 = pltpu.async_copy(src_hbm.at[wid], src_a, sem_i)
        cpi2 = pltpu.async_copy(dst_hbm.at[wid], dst_a, sem_i)
        cpi3 = pltpu.async_copy(hnpk_hbm.at[pl.ds(0, N_TAB)], hnpk_v, sem_i)
        cpi4 = pltpu.async_copy(bvec_hbm, beta_v, sem_i)

        @pl.loop(0, 16)
        def _(i):
            z_v[i, pl.ds(0, 16)] = zero16
            z_v[i, pl.ds(16, 16)] = zero16

        @pl.loop(0, rows_per_tile // 16)
        def _(r):
            pltpu.sync_copy(z_v, acc_sp.at[pl.ds(tid * rows_per_tile + r * 16, 16)])

        @pl.loop(0, B)
        def _(r):
            scaled_v[r, pl.ds(F, F)] = zero16

        cpi1.wait()
        cpi2.wait()
        cpi3.wait()
        cpi4.wait()
        plsc.subcore_barrier()

        def issue(t, b):
            pltpu.async_copy(htab_hbm.at[src_a.at[t]], rows_v.at[b], sem_r[b])

        def wait_gather(t, b):
            pltpu.make_async_copy(
                htab_hbm.at[src_a.at[t]], rows_v.at[b], sem_r[b]).wait()

        def wait_scatter(t):
            pltpu.make_async_copy(
                scaled_v, acc_sp.at[dst_a.at[t]], sem_s).wait()

        def compute(t, b):
            rows = rows_v.at[b]
            scaled = scaled_v
            bv = beta_v[...]

            @plsc.parallel_loop(0, B // 16, unroll=4)
            def _(g):
                r16 = lax.iota(jnp.int32, 16) + g * 16
                s16 = src_a[t, pl.ds(g * 16, 16)]
                d16 = dst_a[t, pl.ds(g * 16, 16)]
                acc = jnp.zeros((16,), jnp.float32)
                for pc in range(F // 2):
                    cpcv = jnp.full((16,), pc, jnp.int32)
                    sp = plsc.load_gather(hnpk_v, [s16, cpcv])
                    dp = plsc.load_gather(hnpk_v, [d16, cpcv])
                    pb = (plsc.bitcast(sp, jnp.bfloat16)
                          * plsc.bitcast(dp, jnp.bfloat16))
                    pe, po = plsc.unpack(pb, format=plsc.PackFormat.INTERLEAVED)
                    acc = acc + (pe + po)
                w = jnp.exp(acc * bv)
                plsc.store_scatter(
                    scaled, [r16, jnp.full((16,), F, jnp.int32)], w)
                for c in range(F):
                    hcomp = plsc.load_gather(
                        rows, [r16, jnp.full((16,), c, jnp.int32)])
                    plsc.store_scatter(
                        scaled, [r16, jnp.full((16,), c, jnp.int32)],
                        hcomp * w)

            pltpu.async_copy(scaled, acc_sp.at[dst_a.at[t]], sem_s,
                             add=True)

        issue(0, 0)

        @pl.loop(0, n_blocks, step=2)
        def _(tt):
            issue(tt + 1, 1)
            wait_gather(tt, 0)

            @pl.when(tt >= 1)
            def _():
                wait_scatter(tt - 1)
            compute(tt, 0)

            @pl.when(tt + 2 < n_blocks)
            def _():
                issue(tt + 2, 0)
            wait_gather(tt + 1, 1)
            wait_scatter(tt)
            compute(tt + 1, 1)

        wait_scatter(n_blocks - 1)
        plsc.subcore_barrier()

        pltpu.sync_copy(
            acc_sp.at[pl.ds(tid * rows_per_tile, rows_per_tile)],
            out_hbm.at[cid, pl.ds(tid * rows_per_tile, rows_per_tile)])

    return k(htab, hnpk, bvec, srcp, dstp)


def kernel(x, edge_index, W1, b1, beta1, beta2, W2, b2):
    n = x.shape[0]
    e_total = edge_index.shape[1] + n
    n_blocks = _num_blocks(e_total)
    e_pad = n_blocks * B * NW

    loop = jnp.arange(n, dtype=edge_index.dtype)
    pad_e = e_pad - e_total
    src = jnp.concatenate([edge_index[0], loop,
                           jnp.zeros((pad_e,), edge_index.dtype)])
    dst = jnp.concatenate([edge_index[1], loop,
                           jnp.full((pad_e,), n, edge_index.dtype)])
    src = src.reshape(NW, n_blocks, B)
    dst = dst.reshape(NW, n_blocks, B)

    x_pad = jnp.pad(x, ((0, N_PAD - n), (0, 0)))
    b1vec = jnp.full((16,), beta1, jnp.float32)
    b2vec = jnp.full((16,), beta2, jnp.float32)

    h1, hn1 = _tc_pre(x_pad, W1, b1)
    part1 = _sc_prop(h1, _pack_pairs(hn1), b1vec, src, dst, n_blocks)
    h2, hn2 = _tc_mid(part1)
    part2 = _sc_prop(h2, _pack_pairs(hn2), b2vec, src, dst, n_blocks)
    return _tc_post(part2, W2, b2, n)


# speedup vs baseline: 43.5953x; 1.2763x over previous
import dataclasses
import functools

import jax
import jax.numpy as jnp
from jax import lax
from jax.experimental import pallas as pl
from jax.experimental.pallas import tpu as pltpu
from jax.experimental.pallas import tpu_sc as plsc

N_PAD = 10240
N_TAB = 10016
F = 16
TW = 32
B = 128
NC, NS = 2, 16
NW = NC * NS
IDXB = 14


def _num_blocks(e_total):
    per_tile = -(-e_total // NW)
    nb = -(-per_tile // B)
    return nb + (nb & 1)


def _tc_pre_body(x_ref, w1_ref, b1_ref, hn_ref, nrm_ref, n_real):
    x = x_ref[...]
    h = jnp.maximum(jnp.dot(x, w1_ref[...],
                            preferred_element_type=jnp.float32,
                            precision=lax.Precision.HIGHEST)
                    + b1_ref[...][None, :], 0.0)
    row = lax.broadcasted_iota(jnp.int32, (x.shape[0], 1), 0)
    h = jnp.where(row < n_real, h, 0.0)
    norm = jnp.sqrt(jnp.sum(h * h, axis=-1, keepdims=True))
    hn = h / jnp.maximum(norm, 1e-12)
    hn_ref[...] = hn
    nrm_ref[...] = norm[:, 0]


def _tc_mid_body(part_ref, hn_ref, nrm_ref):
    feat = part_ref[0, :, 0:F] + part_ref[1, :, 0:F]
    den = part_ref[0, :, F] + part_ref[1, :, F]
    h = feat / (den + 1e-16)[:, None]
    norm = jnp.sqrt(jnp.sum(h * h, axis=-1, keepdims=True))
    hn = h / jnp.maximum(norm, 1e-12)
    hn_ref[...] = hn
    nrm_ref[...] = norm[:, 0]


def _tc_post_body(part_ref, w2_ref, b2_ref, out_ref, n_real):
    feat = part_ref[0, 0:n_real, 0:F] + part_ref[1, 0:n_real, 0:F]
    den = part_ref[0, 0:n_real, F] + part_ref[1, 0:n_real, F]
    h = feat / (den + 1e-16)[:, None]
    logits = jnp.dot(h, w2_ref[...],
                     preferred_element_type=jnp.float32,
                     precision=lax.Precision.HIGHEST) + b2_ref[...][None, :]
    m = jnp.max(logits, axis=-1, keepdims=True)
    z = logits - m
    lse = jnp.log(jnp.sum(jnp.exp(z), axis=-1, keepdims=True))
    out_ref[...] = z - lse


def _tc_pre(x_pad, w1, b1):
    return pl.pallas_call(
        functools.partial(_tc_pre_body, n_real=10000),
        out_shape=(jax.ShapeDtypeStruct((N_PAD, F), jnp.float32),
                   jax.ShapeDtypeStruct((N_PAD,), jnp.float32)),
    )(x_pad, w1, b1)


def _tc_mid(part):
    return pl.pallas_call(
        _tc_mid_body,
        out_shape=(jax.ShapeDtypeStruct((N_PAD, F), jnp.float32),
                   jax.ShapeDtypeStruct((N_PAD,), jnp.float32)),
    )(part)


def _tc_post(part, w2, b2, n_real):
    return pl.pallas_call(
        functools.partial(_tc_post_body, n_real=n_real),
        out_shape=jax.ShapeDtypeStruct((n_real, w2.shape[1]), jnp.float32),
    )(part, w2, b2)


def _pack_pairs(hn):
    hn_bf = hn.astype(jnp.bfloat16)
    return jax.lax.bitcast_convert_type(
        hn_bf.reshape(N_PAD, F // 2, 2), jnp.float32)


def _sc_prop(hnpk, nrm, bvec, pk, n_blocks):
    mesh = plsc.VectorSubcoreMesh(core_axis_name="c", subcore_axis_name="s")
    rows_per_tile = N_PAD // NS

    cp = pltpu.CompilerParams()
    if "needs_layout_passes" in pltpu.CompilerParams.__dataclass_fields__:
        cp = dataclasses.replace(cp, needs_layout_passes=False)
    if "use_tc_tiling_on_sc" in pltpu.CompilerParams.__dataclass_fields__:
        cp = dataclasses.replace(cp, use_tc_tiling_on_sc=False)

    @functools.partial(
        pl.kernel,
        out_type=jax.ShapeDtypeStruct((NC, N_PAD, TW), jnp.float32),
        mesh=mesh,
        compiler_params=cp,
        scratch_types=[
            pltpu.VMEM((n_blocks, B), jnp.int32),
            pltpu.VMEM((N_TAB, F // 2), jnp.float32),
            pltpu.VMEM((N_TAB,), jnp.float32),
            pltpu.VMEM((16,), jnp.float32),
            pltpu.VMEM((2, B), jnp.int32),
            pltpu.VMEM((2, B, TW), jnp.float32),
            pltpu.VMEM((16, TW), jnp.float32),
            pltpu.VMEM_SHARED((N_PAD, TW), jnp.float32),
            pltpu.SemaphoreType.DMA,
            pltpu.SemaphoreType.DMA,
            pltpu.SemaphoreType.DMA,
        ],
    )
    def k(hnpk_hbm, nrm_hbm, bvec_hbm, pk_hbm, out_hbm,
          pk_a, hnpk_v, nrm_v, beta_v, dblk_v, scaled_v, z_v, acc_sp,
          sem_i, sem_s0, sem_s1):
        cid = lax.axis_index("c")
        tid = lax.axis_index("s")
        wid = tid * NC + cid
        sem_s = (sem_s0, sem_s1)

        zero16 = jnp.zeros((16,), jnp.float32)

        cpi1 = pltpu.async_copy(pk_hbm.at[wid], pk_a, sem_i)
        cpi2 = pltpu.async_copy(hnpk_hbm.at[pl.ds(0, N_TAB)], hnpk_v, sem_i)
        cpi3 = pltpu.async_copy(nrm_hbm.at[pl.ds(0, N_TAB)], nrm_v, sem_i)
        cpi4 = pltpu.async_copy(bvec_hbm, beta_v, sem_i)

        @pl.loop(0, 16)
        def _(i):
            z_v[i, pl.ds(0, 16)] = zero16
            z_v[i, pl.ds(16, 16)] = zero16

        @pl.loop(0, rows_per_tile // 16)
        def _(r):
            pltpu.sync_copy(z_v, acc_sp.at[pl.ds(tid * rows_per_tile + r * 16, 16)])

        for b in range(2):
            @pl.loop(0, B)
            def _(r):
                scaled_v.at[b][r, pl.ds(F, F)] = zero16

        cpi1.wait()
        cpi2.wait()
        cpi3.wait()
        cpi4.wait()
        plsc.subcore_barrier()

        def wait_scatter(b):
            pltpu.make_async_copy(
                scaled_v.at[b], acc_sp.at[dblk_v.at[b]], sem_s[b]).wait()

        def compute(t, b):
            scaled = scaled_v.at[b]
            dblk = dblk_v.at[b]
            bv = beta_v[...]

            @plsc.parallel_loop(0, B // 16, unroll=1)
            def _(g):
                r16 = lax.iota(jnp.int32, 16) + g * 16
                pkv = pk_a[t, pl.ds(g * 16, 16)]
                s16 = jnp.bitwise_and(pkv, (1 << IDXB) - 1)
                d16 = lax.shift_right_logical(pkv, IDXB)
                dblk[pl.ds(g * 16, 16)] = d16
                acc = jnp.zeros((16,), jnp.float32)
                comps = []
                for pc in range(F // 2):
                    cpcv = jnp.full((16,), pc, jnp.int32)
                    sp = plsc.load_gather(hnpk_v, [s16, cpcv])
                    dp = plsc.load_gather(hnpk_v, [d16, cpcv])
                    se, so = plsc.unpack(plsc.bitcast(sp, jnp.bfloat16),
                                         format=plsc.PackFormat.INTERLEAVED)
                    de, do = plsc.unpack(plsc.bitcast(dp, jnp.bfloat16),
                                         format=plsc.PackFormat.INTERLEAVED)
                    acc = acc + (se * de + so * do)
                    comps.append((se, so))
                nrm16 = plsc.load_gather(nrm_v, [s16])
                w = jnp.exp(acc * bv)
                wn = w * nrm16
                plsc.store_scatter(
                    scaled, [r16, jnp.full((16,), F, jnp.int32)], w)
                for pc in range(F // 2):
                    se, so = comps[pc]
                    plsc.store_scatter(
                        scaled, [r16, jnp.full((16,), 2 * pc, jnp.int32)],
                        se * wn)
                    plsc.store_scatter(
                        scaled, [r16, jnp.full((16,), 2 * pc + 1, jnp.int32)],
                        so * wn)

            pltpu.async_copy(scaled, acc_sp.at[dblk], sem_s[b], add=True)

        @pl.loop(0, n_blocks, step=2)
        def _(tt):
            @pl.when(tt >= 2)
            def _():
                wait_scatter(0)
            compute(tt, 0)

            @pl.when(tt >= 2)
            def _():
                wait_scatter(1)
            compute(tt + 1, 1)

        wait_scatter(0)
        wait_scatter(1)
        plsc.subcore_barrier()

        pltpu.sync_copy(
            acc_sp.at[pl.ds(tid * rows_per_tile, rows_per_tile)],
            out_hbm.at[cid, pl.ds(tid * rows_per_tile, rows_per_tile)])

    return k(hnpk, nrm, bvec, pk)


def kernel(x, edge_index, W1, b1, beta1, beta2, W2, b2):
    n = x.shape[0]
    e_total = edge_index.shape[1] + n
    n_blocks = _num_blocks(e_total)
    e_pad = n_blocks * B * NW

    loop = jnp.arange(n, dtype=edge_index.dtype)
    pad_e = e_pad - e_total
    src = jnp.concatenate([edge_index[0], loop,
                           jnp.zeros((pad_e,), edge_index.dtype)])
    dst = jnp.concatenate([edge_index[1], loop,
                           jnp.full((pad_e,), n, edge_index.dtype)])
    pk = jnp.bitwise_or(src, jnp.left_shift(dst, IDXB))
    pk = pk.reshape(NW, n_blocks, B)

    x_pad = jnp.pad(x, ((0, N_PAD - n), (0, 0)))
    b1vec = jnp.full((16,), beta1, jnp.float32)
    b2vec = jnp.full((16,), beta2, jnp.float32)

    hn1, nrm1 = _tc_pre(x_pad, W1, b1)
    part1 = _sc_prop(_pack_pairs(hn1), nrm1, b1vec, pk, n_blocks)
    hn2, nrm2 = _tc_mid(part1)
    part2 = _sc_prop(_pack_pairs(hn2), nrm2, b2vec, pk, n_blocks)
    return _tc_post(part2, W2, b2, n)


# speedup vs baseline: 43.6161x; 1.0005x over previous
import dataclasses
import functools

import jax
import jax.numpy as jnp
from jax import lax
from jax.experimental import pallas as pl
from jax.experimental.pallas import tpu as pltpu
from jax.experimental.pallas import tpu_sc as plsc

N_PAD = 10240
N_TAB = 10016
F = 16
TW = 32
B = 128
NC, NS = 2, 16
NW = NC * NS
IDXB = 14


def _num_blocks(e_total):
    per_tile = -(-e_total // NW)
    nb = -(-per_tile // B)
    return nb + (nb & 1)


def _tc_pre_body(x_ref, w1_ref, b1_ref, hn_ref, nrm_ref, n_real):
    x = x_ref[...]
    h = jnp.maximum(jnp.dot(x, w1_ref[...],
                            preferred_element_type=jnp.float32,
                            precision=lax.Precision.HIGHEST)
                    + b1_ref[...][None, :], 0.0)
    row = lax.broadcasted_iota(jnp.int32, (x.shape[0], 1), 0)
    h = jnp.where(row < n_real, h, 0.0)
    norm = jnp.sqrt(jnp.sum(h * h, axis=-1, keepdims=True))
    hn = h / jnp.maximum(norm, 1e-12)
    hn_ref[...] = hn
    nrm_ref[...] = norm[:, 0]


def _tc_mid_body(part_ref, hn_ref, nrm_ref):
    feat = part_ref[0, :, 0:F] + part_ref[1, :, 0:F]
    den = part_ref[0, :, F] + part_ref[1, :, F]
    h = feat / (den + 1e-16)[:, None]
    norm = jnp.sqrt(jnp.sum(h * h, axis=-1, keepdims=True))
    hn = h / jnp.maximum(norm, 1e-12)
    hn_ref[...] = hn
    nrm_ref[...] = norm[:, 0]


def _tc_post_body(part_ref, w2_ref, b2_ref, out_ref, n_real):
    feat = part_ref[0, 0:n_real, 0:F] + part_ref[1, 0:n_real, 0:F]
    den = part_ref[0, 0:n_real, F] + part_ref[1, 0:n_real, F]
    h = feat / (den + 1e-16)[:, None]
    logits = jnp.dot(h, w2_ref[...],
                     preferred_element_type=jnp.float32,
                     precision=lax.Precision.HIGHEST) + b2_ref[...][None, :]
    m = jnp.max(logits, axis=-1, keepdims=True)
    z = logits - m
    lse = jnp.log(jnp.sum(jnp.exp(z), axis=-1, keepdims=True))
    out_ref[...] = z - lse


def _tc_pre(x_pad, w1, b1):
    return pl.pallas_call(
        functools.partial(_tc_pre_body, n_real=10000),
        out_shape=(jax.ShapeDtypeStruct((N_PAD, F), jnp.float32),
                   jax.ShapeDtypeStruct((N_PAD,), jnp.float32)),
    )(x_pad, w1, b1)


def _tc_mid(part):
    return pl.pallas_call(
        _tc_mid_body,
        out_shape=(jax.ShapeDtypeStruct((N_PAD, F), jnp.float32),
                   jax.ShapeDtypeStruct((N_PAD,), jnp.float32)),
    )(part)


def _tc_post(part, w2, b2, n_real):
    return pl.pallas_call(
        functools.partial(_tc_post_body, n_real=n_real),
        out_shape=jax.ShapeDtypeStruct((n_real, w2.shape[1]), jnp.float32),
    )(part, w2, b2)


def _pack_pairs(hn):
    hn_bf = hn.astype(jnp.bfloat16)
    return jax.lax.bitcast_convert_type(
        hn_bf.reshape(N_PAD, F // 2, 2), jnp.float32)


def _sc_prop(hnpk, nrm, bvec, pk, n_blocks):
    mesh = plsc.VectorSubcoreMesh(core_axis_name="c", subcore_axis_name="s")
    rows_per_tile = N_PAD // NS

    cp = pltpu.CompilerParams()
    if "needs_layout_passes" in pltpu.CompilerParams.__dataclass_fields__:
        cp = dataclasses.replace(cp, needs_layout_passes=False)
    if "use_tc_tiling_on_sc" in pltpu.CompilerParams.__dataclass_fields__:
        cp = dataclasses.replace(cp, use_tc_tiling_on_sc=False)

    @functools.partial(
        pl.kernel,
        out_type=jax.ShapeDtypeStruct((NC, N_PAD, TW), jnp.float32),
        mesh=mesh,
        compiler_params=cp,
        scratch_types=[
            pltpu.VMEM((n_blocks, B), jnp.int32),
            pltpu.VMEM((N_TAB, F // 2), jnp.float32),
            pltpu.VMEM((N_TAB,), jnp.float32),
            pltpu.VMEM((16,), jnp.float32),
            pltpu.VMEM((2, B), jnp.int32),
            pltpu.VMEM((2, B, TW), jnp.float32),
            pltpu.VMEM_SHARED((N_PAD, TW), jnp.float32),
            pltpu.SemaphoreType.DMA,
            pltpu.SemaphoreType.DMA,
            pltpu.SemaphoreType.DMA,
        ],
    )
    def k(hnpk_hbm, nrm_hbm, bvec_hbm, pk_hbm, out_hbm,
          pk_a, hnpk_v, nrm_v, beta_v, dblk_v, scaled_v, acc_sp,
          sem_i, sem_s0, sem_s1):
        cid = lax.axis_index("c")
        tid = lax.axis_index("s")
        wid = tid * NC + cid
        sem_s = (sem_s0, sem_s1)

        zero16 = jnp.zeros((16,), jnp.float32)

        cpi1 = pltpu.async_copy(pk_hbm.at[wid], pk_a, sem_i)
        cpi2 = pltpu.async_copy(hnpk_hbm.at[pl.ds(0, N_TAB)], hnpk_v, sem_i)
        cpi3 = pltpu.async_copy(nrm_hbm.at[pl.ds(0, N_TAB)], nrm_v, sem_i)
        cpi4 = pltpu.async_copy(bvec_hbm, beta_v, sem_i)

        for b in range(2):
            @pl.loop(0, B)
            def _(r):
                scaled_v.at[b][r, pl.ds(0, 16)] = zero16
                scaled_v.at[b][r, pl.ds(16, 16)] = zero16

        for i in range(rows_per_tile // B):
            pltpu.sync_copy(scaled_v.at[i % 2],
                            acc_sp.at[pl.ds(tid * rows_per_tile + i * B, B)])

        cpi1.wait()
        cpi2.wait()
        cpi3.wait()
        cpi4.wait()
        plsc.subcore_barrier()

        def wait_scatter(b):
            pltpu.make_async_copy(
                scaled_v.at[b], acc_sp.at[dblk_v.at[b]], sem_s[b]).wait()

        def compute(t, b):
            scaled = scaled_v.at[b]
            dblk = dblk_v.at[b]
            bv = beta_v[...]

            @plsc.parallel_loop(0, B // 16, unroll=1)
            def _(g):
                r16 = lax.iota(jnp.int32, 16) + g * 16
                pkv = pk_a[t, pl.ds(g * 16, 16)]
                s16 = jnp.bitwise_and(pkv, (1 << IDXB) - 1)
                d16 = lax.shift_right_logical(pkv, IDXB)
                dblk[pl.ds(g * 16, 16)] = d16
                acc = jnp.zeros((16,), jnp.float32)
                comps = []
                for pc in range(F // 2):
                    cpcv = jnp.full((16,), pc, jnp.int32)
                    sp = plsc.load_gather(hnpk_v, [s16, cpcv])
                    dp = plsc.load_gather(hnpk_v, [d16, cpcv])
                    se, so = plsc.unpack(plsc.bitcast(sp, jnp.bfloat16),
                                         format=plsc.PackFormat.INTERLEAVED)
                    de, do = plsc.unpack(plsc.bitcast(dp, jnp.bfloat16),
                                         format=plsc.PackFormat.INTERLEAVED)
                    acc = acc + (se * de + so * do)
                    comps.append((se, so))
                nrm16 = plsc.load_gather(nrm_v, [s16])
                w = jnp.exp(acc * bv)
                wn = w * nrm16
                plsc.store_scatter(
                    scaled, [r16, jnp.full((16,), F, jnp.int32)], w)
                for pc in range(F // 2):
                    se, so = comps[pc]
                    plsc.store_scatter(
                        scaled, [r16, jnp.full((16,), 2 * pc, jnp.int32)],
                        se * wn)
                    plsc.store_scatter(
                        scaled, [r16, jnp.full((16,), 2 * pc + 1, jnp.int32)],
                        so * wn)

            pltpu.async_copy(scaled, acc_sp.at[dblk], sem_s[b], add=True)

        @pl.loop(0, n_blocks, step=2)
        def _(tt):
            @pl.when(tt >= 2)
            def _():
                wait_scatter(0)
            compute(tt, 0)

            @pl.when(tt >= 2)
            def _():
                wait_scatter(1)
            compute(tt + 1, 1)

        wait_scatter(0)
        wait_scatter(1)
        plsc.subcore_barrier()

        pltpu.sync_copy(
            acc_sp.at[pl.ds(tid * rows_per_tile, rows_per_tile)],
            out_hbm.at[cid, pl.ds(tid * rows_per_tile, rows_per_tile)])

    return k(hnpk, nrm, bvec, pk)


def kernel(x, edge_index, W1, b1, beta1, beta2, W2, b2):
    n = x.shape[0]
    e_total = edge_index.shape[1] + n
    n_blocks = _num_blocks(e_total)
    e_pad = n_blocks * B * NW

    loop = jnp.arange(n, dtype=edge_index.dtype)
    pad_e = e_pad - e_total
    src = jnp.concatenate([edge_index[0], loop,
                           jnp.zeros((pad_e,), edge_index.dtype)])
    dst = jnp.concatenate([edge_index[1], loop,
                           jnp.full((pad_e,), n, edge_index.dtype)])
    pk = jnp.bitwise_or(src, jnp.left_shift(dst, IDXB))
    pk = pk.reshape(NW, n_blocks, B)

    x_pad = jnp.pad(x, ((0, N_PAD - n), (0, 0)))
    b1vec = jnp.full((16,), beta1, jnp.float32)
    b2vec = jnp.full((16,), beta2, jnp.float32)

    hn1, nrm1 = _tc_pre(x_pad, W1, b1)
    part1 = _sc_prop(_pack_pairs(hn1), nrm1, b1vec, pk, n_blocks)
    hn2, nrm2 = _tc_mid(part1)
    part2 = _sc_prop(_pack_pairs(hn2), nrm2, b2vec, pk, n_blocks)
    return _tc_post(part2, W2, b2, n)
